# baseline (speedup 1.0000x reference)
import functools

import jax
import jax.numpy as jnp
from jax import lax
from jax.experimental import pallas as pl
from jax.experimental.pallas import tpu as pltpu
from jax.experimental.pallas import tpu_sc as plsc

FEATURE_DIM = 1000000
EMBED_DIM = 16
BATCH = 16384
N_FIELDS = 26
TOTAL = BATCH * N_FIELDS

NUM_CORES = 2
NUM_SUBCORES = 16
NUM_WORKERS = NUM_CORES * NUM_SUBCORES

ROWS_MAIN = 999936
A_CHUNK = 1536
A_NCHUNKS = ROWS_MAIN // A_CHUNK
A_TMAX = 22
A_LINES = A_CHUNK // 16
SCR_LINES = FEATURE_DIM // 16

B_CHUNK = 256
PER_WORKER = TOTAL // NUM_WORKERS
B_NCHUNKS = PER_WORKER // B_CHUNK


def _iota16():
    return lax.iota(jnp.int32, 16)


@functools.cache
def _build_transpose():
    mesh = plsc.VectorSubcoreMesh(core_axis_name="c", subcore_axis_name="s")

    @functools.partial(
        pl.kernel,
        mesh=mesh,
        compiler_params=pltpu.CompilerParams(needs_layout_passes=False),
        out_type=jax.ShapeDtypeStruct((SCR_LINES, 128), jnp.float32),
        scratch_types=[
            pltpu.VMEM((16, A_CHUNK), jnp.float32),
            pltpu.VMEM((16, A_CHUNK), jnp.float32),
            pltpu.VMEM((A_LINES, 128), jnp.float32),
            pltpu.VMEM((A_LINES, 128), jnp.float32),
            pltpu.VMEM((4, 128), jnp.float32),
            pltpu.SemaphoreType.DMA,
            pltpu.SemaphoreType.DMA,
            pltpu.SemaphoreType.DMA,
            pltpu.SemaphoreType.DMA,
        ],
    )
    def transpose_kernel(wt_hbm, wtail_hbm, scr_hbm,
                         in0, in1, ob0, ob1, tailb,
                         isem0, isem1, osem0, osem1):
        wid = lax.axis_index("s") * NUM_CORES + lax.axis_index("c")
        ins = (in0, in1)
        obs = (ob0, ob1)
        isems = (isem0, isem1)
        osems = (osem0, osem1)
        iota = _iota16()

        def chunk_id(t):
            return wid + NUM_WORKERS * t

        def issue_in(t, p):
            c = chunk_id(t)

            @pl.when(c < A_NCHUNKS)
            def _():
                off = pl.multiple_of(c * A_CHUNK, A_CHUNK)
                pltpu.async_copy(
                    wt_hbm.at[:, pl.ds(off, A_CHUNK)], ins[p], isems[p])

        def process(t, p, first):
            c = chunk_id(t)

            @pl.when(c < A_NCHUNKS)
            def _():
                l0 = pl.multiple_of(c * A_LINES, A_LINES)
                dst = scr_hbm.at[pl.ds(l0, A_LINES), :]
                if not first:
                    pltpu.make_async_copy(obs[p], dst, osems[p]).wait()
                pltpu.make_async_copy(
                    wt_hbm.at[:, pl.ds(0, A_CHUNK)], ins[p], isems[p]).wait()

                for j in range(8):
                    q = jax.lax.bitwise_and(iota + j, 15)
                    q8 = jax.lax.bitwise_and(iota + j + 8, 15)
                    is_lo = q < 8
                    cc = jax.lax.shift_left(jax.lax.bitwise_and(q, 7),
                                            4) + iota

                    def _blk(i, carry):
                        gcol, gcol8 = carry
                        va = plsc.load_gather(ins[p], [iota, gcol])
                        vb = plsc.load_gather(ins[p], [iota, gcol8])
                        lo = jnp.where(is_lo, va, vb)
                        hi = jnp.where(is_lo, vb, va)
                        packed = plsc.bitcast(
                            plsc.pack(lo, hi,
                                      format=plsc.PackFormat.INTERLEAVED),
                            jnp.float32)
                        plsc.store_scatter(
                            obs[p], [jnp.full((16,), i, jnp.int32), cc],
                            packed)
                        return (gcol + 16, gcol8 + 16)

                    plsc.parallel_loop(0, A_CHUNK // 16, 1, unroll=8,
                                       carry=(q, q8))(_blk)
                pltpu.async_copy(obs[p], dst, osems[p])
                issue_in(t + 2, p)

        @pl.when(wid == 0)
        def _():
            pltpu.sync_copy(wtail_hbm, tailb)
            pltpu.sync_copy(tailb, scr_hbm.at[pl.ds(SCR_LINES - 4, 4), :])

        issue_in(0, 0)
        issue_in(1, 1)
        process(0, 0, True)
        process(1, 1, True)

        def outer(tt, carry):
            process(2 * tt, 0, False)
            process(2 * tt + 1, 1, False)
            return carry

        lax.fori_loop(1, A_TMAX // 2, outer, 0)

        dst0 = scr_hbm.at[pl.ds(0, A_LINES), :]
        pltpu.make_async_copy(ob0, dst0, osem0).wait()
        pltpu.make_async_copy(ob1, dst0, osem1).wait()

    return transpose_kernel


@functools.cache
def _build_gather():
    mesh = plsc.VectorSubcoreMesh(core_axis_name="c", subcore_axis_name="s")

    @functools.partial(
        pl.kernel,
        mesh=mesh,
        compiler_params=pltpu.CompilerParams(needs_layout_passes=False),
        out_type=jax.ShapeDtypeStruct((N_FIELDS, EMBED_DIM, BATCH),
                                      jnp.float32),
        scratch_types=[
            pltpu.VMEM((PER_WORKER,), jnp.int32),
            pltpu.VMEM((PER_WORKER,), jnp.int32),
            pltpu.VMEM((PER_WORKER,), jnp.int32),
            pltpu.VMEM((PER_WORKER,), jnp.int32),
            pltpu.VMEM((B_CHUNK, 128), jnp.float32),
            pltpu.VMEM((B_CHUNK, 128), jnp.float32),
            pltpu.VMEM((16, B_CHUNK), jnp.float32),
            pltpu.VMEM((16, B_CHUNK), jnp.float32),
            pltpu.SemaphoreType.DMA,
            pltpu.SemaphoreType.DMA,
            pltpu.SemaphoreType.DMA,
            pltpu.SemaphoreType.DMA,
        ],
    )
    def gather_kernel(scr_hbm, idx_hbm, out_hbm,
                      idxs_v, rv_all, sv_all, hv_all, st0, st1, ot0, ot1,
                      gsem0, gsem1, osem0, osem1):
        wid = lax.axis_index("s") * NUM_CORES + lax.axis_index("c")
        qbase = wid * PER_WORKER
        stages = (st0, st1)
        outs = (ot0, ot1)
        gsems = (gsem0, gsem1)
        osems = (osem0, osem1)
        iota = _iota16()

        pltpu.sync_copy(idx_hbm.at[pl.ds(qbase, PER_WORKER)], idxs_v)

        def _prep(k):
            v = idxs_v[pl.ds(k * 16, 16)]
            rv_all[pl.ds(k * 16, 16)] = jax.lax.shift_right_logical(v, 4)
            sv_all[pl.ds(k * 16, 16)] = jax.lax.shift_left(
                jax.lax.bitwise_and(v, 7), 4)
            hv_all[pl.ds(k * 16, 16)] = jax.lax.shift_left(
                jax.lax.bitwise_and(jax.lax.shift_right_logical(v, 3), 1),
                4)

        plsc.parallel_loop(0, PER_WORKER // 16, 1, unroll=4)(_prep)

        def fire(t, p):
            @pl.when(t < B_NCHUNKS)
            def _():
                toff = pl.multiple_of(t * B_CHUNK, B_CHUNK)
                pltpu.async_copy(
                    scr_hbm.at[rv_all.at[pl.ds(toff, B_CHUNK)]],
                    stages[p], gsems[p])

        def out_slab(t):
            q0 = qbase + t * B_CHUNK
            f = jax.lax.shift_right_logical(q0, 14)
            b0 = pl.multiple_of(jax.lax.bitwise_and(q0, BATCH - 1), B_CHUNK)
            return out_hbm.at[f, :, pl.ds(b0, B_CHUNK)]

        def process(t, p, first):
            dst = out_slab(t)
            if not first:
                pltpu.make_async_copy(outs[p], dst, osems[p]).wait()
            pltpu.make_async_copy(
                scr_hbm.at[rv_all.at[pl.ds(0, B_CHUNK)]],
                stages[p], gsems[p]).wait()

            toffv = jnp.full((16,), t * B_CHUNK, jnp.int32)
            for j in range(16):
                q = jax.lax.bitwise_and(iota + j, 15)

                def _blk(i, rv):
                    g = rv + toffv
                    sv = plsc.load_gather(sv_all, [g])
                    hv = plsc.load_gather(hv_all, [g])
                    w = plsc.load_gather(stages[p], [rv, sv + iota])
                    wi = plsc.bitcast(w, jnp.int32)
                    vals = plsc.bitcast(
                        jax.lax.shift_left(
                            jax.lax.shift_right_logical(wi, hv), 16),
                        jnp.float32)
                    plsc.store_scatter(outs[p], [iota, rv], vals)
                    return rv + 16

                plsc.parallel_loop(0, B_CHUNK // 16, 1, unroll=8,
                                   carry=q)(_blk)
            pltpu.async_copy(outs[p], dst, osems[p])
            fire(t + 2, p)

        fire(0, 0)
        fire(1, 1)
        process(0, 0, True)
        process(1, 1, True)

        def outer(tt, carry):
            process(2 * tt, 0, False)
            process(2 * tt + 1, 1, False)
            return carry

        lax.fori_loop(1, B_NCHUNKS // 2, outer, 0)

        dst0 = out_hbm.at[0, :, pl.ds(0, B_CHUNK)]
        pltpu.make_async_copy(ot0, dst0, osem0).wait()
        pltpu.make_async_copy(ot1, dst0, osem1).wait()

    return gather_kernel


def kernel(x, W):
    wt = W.T
    wtail = lax.slice(W, (ROWS_MAIN, 0), (FEATURE_DIM, EMBED_DIM))
    wtail = wtail.astype(jnp.bfloat16).reshape(4, 2, 8, EMBED_DIM)
    wtail = jnp.transpose(wtail, (0, 2, 3, 1))
    wtail = jax.lax.bitcast_convert_type(wtail, jnp.float32)
    wtail = wtail.reshape(4, 128)
    idx = x.T.reshape(TOTAL).astype(jnp.int32)
    w_scr = _build_transpose()(wt, wtail)
    out3 = _build_gather()(w_scr, idx)
    return jnp.transpose(out3, (2, 0, 1))


# speedup vs baseline: 1.0674x; 1.0674x over previous
import functools

import jax
import jax.numpy as jnp
from jax import lax
from jax.experimental import pallas as pl
from jax.experimental.pallas import tpu as pltpu
from jax.experimental.pallas import tpu_sc as plsc

FEATURE_DIM = 1000000
EMBED_DIM = 16
BATCH = 16384
N_FIELDS = 26
TOTAL = BATCH * N_FIELDS

NUM_CORES = 2
NUM_SUBCORES = 16
NUM_WORKERS = NUM_CORES * NUM_SUBCORES

ROWS_MAIN = 999936
A_CHUNK = 1536
A_NCHUNKS = ROWS_MAIN // A_CHUNK
A_TMAX = 22
A_LINES = A_CHUNK // 16
SCR_LINES = FEATURE_DIM // 16

B_CHUNK = 256
PER_WORKER = TOTAL // NUM_WORKERS
B_NCHUNKS = PER_WORKER // B_CHUNK


def _iota16():
    return lax.iota(jnp.int32, 16)


@functools.cache
def _build_transpose():
    mesh = plsc.VectorSubcoreMesh(core_axis_name="c", subcore_axis_name="s")

    @functools.partial(
        pl.kernel,
        mesh=mesh,
        compiler_params=pltpu.CompilerParams(needs_layout_passes=False),
        out_type=jax.ShapeDtypeStruct((SCR_LINES, 128), jnp.float32),
        scratch_types=[
            pltpu.VMEM((16, A_CHUNK), jnp.float32),
            pltpu.VMEM((16, A_CHUNK), jnp.float32),
            pltpu.VMEM((A_LINES, 128), jnp.float32),
            pltpu.VMEM((A_LINES, 128), jnp.float32),
            pltpu.VMEM((4, 128), jnp.float32),
            pltpu.SemaphoreType.DMA,
            pltpu.SemaphoreType.DMA,
            pltpu.SemaphoreType.DMA,
            pltpu.SemaphoreType.DMA,
        ],
    )
    def transpose_kernel(wt_hbm, wtail_hbm, scr_hbm,
                         in0, in1, ob0, ob1, tailb,
                         isem0, isem1, osem0, osem1):
        wid = lax.axis_index("s") * NUM_CORES + lax.axis_index("c")
        ins = (in0, in1)
        obs = (ob0, ob1)
        isems = (isem0, isem1)
        osems = (osem0, osem1)
        iota = _iota16()

        def chunk_id(t):
            return wid + NUM_WORKERS * t

        def issue_in(t, p):
            c = chunk_id(t)

            @pl.when(c < A_NCHUNKS)
            def _():
                off = pl.multiple_of(c * A_CHUNK, A_CHUNK)
                pltpu.async_copy(
                    wt_hbm.at[:, pl.ds(off, A_CHUNK)], ins[p], isems[p])

        def process(t, p, first):
            c = chunk_id(t)

            @pl.when(c < A_NCHUNKS)
            def _():
                l0 = pl.multiple_of(c * A_LINES, A_LINES)
                dst = scr_hbm.at[pl.ds(l0, A_LINES), :]
                if not first:
                    pltpu.make_async_copy(obs[p], dst, osems[p]).wait()
                pltpu.make_async_copy(
                    wt_hbm.at[:, pl.ds(0, A_CHUNK)], ins[p], isems[p]).wait()

                for j in range(8):
                    q = jax.lax.bitwise_and(iota + j, 15)
                    q8 = jax.lax.bitwise_and(iota + j + 8, 15)
                    is_lo = q < 8
                    cc = jax.lax.shift_left(jax.lax.bitwise_and(q, 7),
                                            4) + iota

                    def _blk(i, carry):
                        gcol, gcol8 = carry
                        va = plsc.load_gather(ins[p], [iota, gcol])
                        vb = plsc.load_gather(ins[p], [iota, gcol8])
                        lo = jnp.where(is_lo, va, vb)
                        hi = jnp.where(is_lo, vb, va)
                        packed = plsc.bitcast(
                            plsc.pack(lo, hi,
                                      format=plsc.PackFormat.INTERLEAVED),
                            jnp.float32)
                        plsc.store_scatter(
                            obs[p], [jnp.full((16,), i, jnp.int32), cc],
                            packed)
                        return (gcol + 16, gcol8 + 16)

                    plsc.parallel_loop(0, A_CHUNK // 16, 1, unroll=4,
                                       carry=(q, q8))(_blk)
                pltpu.async_copy(obs[p], dst, osems[p])
                issue_in(t + 2, p)

        @pl.when(wid == 0)
        def _():
            pltpu.sync_copy(wtail_hbm, tailb)
            pltpu.sync_copy(tailb, scr_hbm.at[pl.ds(SCR_LINES - 4, 4), :])

        issue_in(0, 0)
        issue_in(1, 1)
        process(0, 0, True)
        process(1, 1, True)

        def outer(tt, carry):
            process(2 * tt, 0, False)
            process(2 * tt + 1, 1, False)
            return carry

        lax.fori_loop(1, A_TMAX // 2, outer, 0)

        dst0 = scr_hbm.at[pl.ds(0, A_LINES), :]
        pltpu.make_async_copy(ob0, dst0, osem0).wait()
        pltpu.make_async_copy(ob1, dst0, osem1).wait()

    return transpose_kernel


@functools.cache
def _build_gather():
    mesh = plsc.VectorSubcoreMesh(core_axis_name="c", subcore_axis_name="s")

    @functools.partial(
        pl.kernel,
        mesh=mesh,
        compiler_params=pltpu.CompilerParams(needs_layout_passes=False),
        out_type=jax.ShapeDtypeStruct((N_FIELDS, EMBED_DIM, BATCH),
                                      jnp.float32),
        scratch_types=[
            pltpu.VMEM((PER_WORKER,), jnp.int32),
            pltpu.VMEM((PER_WORKER,), jnp.int32),
            pltpu.VMEM((PER_WORKER,), jnp.int32),
            pltpu.VMEM((PER_WORKER,), jnp.int32),
            pltpu.VMEM((B_CHUNK, 128), jnp.float32),
            pltpu.VMEM((B_CHUNK, 128), jnp.float32),
            pltpu.VMEM((16, B_CHUNK), jnp.float32),
            pltpu.VMEM((16, B_CHUNK), jnp.float32),
            pltpu.SemaphoreType.DMA,
            pltpu.SemaphoreType.DMA,
            pltpu.SemaphoreType.DMA,
            pltpu.SemaphoreType.DMA,
        ],
    )
    def gather_kernel(scr_hbm, idx_hbm, out_hbm,
                      idxs_v, rv_all, sv_all, hv_all, st0, st1, ot0, ot1,
                      gsem0, gsem1, osem0, osem1):
        wid = lax.axis_index("s") * NUM_CORES + lax.axis_index("c")
        qbase = wid * PER_WORKER
        stages = (st0, st1)
        outs = (ot0, ot1)
        gsems = (gsem0, gsem1)
        osems = (osem0, osem1)
        iota = _iota16()

        pltpu.sync_copy(idx_hbm.at[pl.ds(qbase, PER_WORKER)], idxs_v)

        def _prep(k):
            v = idxs_v[pl.ds(k * 16, 16)]
            rv_all[pl.ds(k * 16, 16)] = jax.lax.shift_right_logical(v, 4)
            sv_all[pl.ds(k * 16, 16)] = jax.lax.shift_left(
                jax.lax.bitwise_and(v, 7), 4)
            hv_all[pl.ds(k * 16, 16)] = jax.lax.shift_left(
                jax.lax.bitwise_and(jax.lax.shift_right_logical(v, 3), 1),
                4)

        plsc.parallel_loop(0, PER_WORKER // 16, 1, unroll=4)(_prep)

        def fire(t, p):
            @pl.when(t < B_NCHUNKS)
            def _():
                toff = pl.multiple_of(t * B_CHUNK, B_CHUNK)
                h = B_CHUNK // 2
                pltpu.async_copy(
                    scr_hbm.at[rv_all.at[pl.ds(toff, h)]],
                    stages[p].at[pl.ds(0, h), :], gsems[p])
                pltpu.async_copy(
                    scr_hbm.at[rv_all.at[pl.ds(toff + h, h)]],
                    stages[p].at[pl.ds(h, h), :], gsems[p])

        def out_slab(t):
            q0 = qbase + t * B_CHUNK
            f = jax.lax.shift_right_logical(q0, 14)
            b0 = pl.multiple_of(jax.lax.bitwise_and(q0, BATCH - 1), B_CHUNK)
            return out_hbm.at[f, :, pl.ds(b0, B_CHUNK)]

        def process(t, p, first):
            dst = out_slab(t)
            if not first:
                pltpu.make_async_copy(outs[p], dst, osems[p]).wait()
            pltpu.make_async_copy(
                scr_hbm.at[rv_all.at[pl.ds(0, B_CHUNK)]],
                stages[p], gsems[p]).wait()

            toffv = jnp.full((16,), t * B_CHUNK, jnp.int32)
            for j in range(16):
                q = jax.lax.bitwise_and(iota + j, 15)

                def _blk(i, rv):
                    g = rv + toffv
                    sv = plsc.load_gather(sv_all, [g])
                    hv = plsc.load_gather(hv_all, [g])
                    w = plsc.load_gather(stages[p], [rv, sv + iota])
                    wi = plsc.bitcast(w, jnp.int32)
                    vals = plsc.bitcast(
                        jax.lax.shift_left(
                            jax.lax.shift_right_logical(wi, hv), 16),
                        jnp.float32)
                    plsc.store_scatter(outs[p], [iota, rv], vals)
                    return rv + 16

                plsc.parallel_loop(0, B_CHUNK // 16, 1, unroll=4,
                                   carry=q)(_blk)
            pltpu.async_copy(outs[p], dst, osems[p])
            fire(t + 2, p)

        fire(0, 0)
        fire(1, 1)
        process(0, 0, True)
        process(1, 1, True)

        def outer(tt, carry):
            process(2 * tt, 0, False)
            process(2 * tt + 1, 1, False)
            return carry

        lax.fori_loop(1, B_NCHUNKS // 2, outer, 0)

        dst0 = out_hbm.at[0, :, pl.ds(0, B_CHUNK)]
        pltpu.make_async_copy(ot0, dst0, osem0).wait()
        pltpu.make_async_copy(ot1, dst0, osem1).wait()

    return gather_kernel


def kernel(x, W):
    wt = W.T
    wtail = lax.slice(W, (ROWS_MAIN, 0), (FEATURE_DIM, EMBED_DIM))
    wtail = wtail.astype(jnp.bfloat16).reshape(4, 2, 8, EMBED_DIM)
    wtail = jnp.transpose(wtail, (0, 2, 3, 1))
    wtail = jax.lax.bitcast_convert_type(wtail, jnp.float32)
    wtail = wtail.reshape(4, 128)
    idx = x.T.reshape(TOTAL).astype(jnp.int32)
    w_scr = _build_transpose()(wt, wtail)
    out3 = _build_gather()(w_scr, idx)
    return jnp.transpose(out3, (2, 0, 1))


# speedup vs baseline: 1.0854x; 1.0169x over previous
import functools

import jax
import jax.numpy as jnp
from jax import lax
from jax.experimental import pallas as pl
from jax.experimental.pallas import tpu as pltpu
from jax.experimental.pallas import tpu_sc as plsc

FEATURE_DIM = 1000000
EMBED_DIM = 16
BATCH = 16384
N_FIELDS = 26
TOTAL = BATCH * N_FIELDS

NUM_CORES = 2
NUM_SUBCORES = 16
NUM_WORKERS = NUM_CORES * NUM_SUBCORES

ROWS_MAIN = 999936
A_CHUNK = 1536
A_NCHUNKS = ROWS_MAIN // A_CHUNK
A_TMAX = 22
A_LINES = A_CHUNK // 16
SCR_LINES = FEATURE_DIM // 16

B_CHUNK = 256
PER_WORKER = TOTAL // NUM_WORKERS
B_NCHUNKS = PER_WORKER // B_CHUNK


def _iota16():
    return lax.iota(jnp.int32, 16)


@functools.cache
def _build_transpose():
    mesh = plsc.VectorSubcoreMesh(core_axis_name="c", subcore_axis_name="s")

    @functools.partial(
        pl.kernel,
        mesh=mesh,
        compiler_params=pltpu.CompilerParams(needs_layout_passes=False),
        out_type=jax.ShapeDtypeStruct((SCR_LINES, 128), jnp.float32),
        scratch_types=[
            pltpu.VMEM((16, A_CHUNK), jnp.float32),
            pltpu.VMEM((16, A_CHUNK), jnp.float32),
            pltpu.VMEM((A_LINES, 128), jnp.float32),
            pltpu.VMEM((A_LINES, 128), jnp.float32),
            pltpu.VMEM((4, 128), jnp.float32),
            pltpu.SemaphoreType.DMA,
            pltpu.SemaphoreType.DMA,
            pltpu.SemaphoreType.DMA,
            pltpu.SemaphoreType.DMA,
        ],
    )
    def transpose_kernel(wt_hbm, wtail_hbm, scr_hbm,
                         in0, in1, ob0, ob1, tailb,
                         isem0, isem1, osem0, osem1):
        wid = lax.axis_index("s") * NUM_CORES + lax.axis_index("c")
        ins = (in0, in1)
        obs = (ob0, ob1)
        isems = (isem0, isem1)
        osems = (osem0, osem1)
        iota = _iota16()

        def chunk_id(t):
            return wid + NUM_WORKERS * t

        def issue_in(t, p):
            c = chunk_id(t)

            @pl.when(c < A_NCHUNKS)
            def _():
                off = pl.multiple_of(c * A_CHUNK, A_CHUNK)
                pltpu.async_copy(
                    wt_hbm.at[:, pl.ds(off, A_CHUNK)], ins[p], isems[p])

        def process(t, p, first):
            c = chunk_id(t)

            @pl.when(c < A_NCHUNKS)
            def _():
                l0 = pl.multiple_of(c * A_LINES, A_LINES)
                dst = scr_hbm.at[pl.ds(l0, A_LINES), :]
                if not first:
                    pltpu.make_async_copy(obs[p], dst, osems[p]).wait()
                pltpu.make_async_copy(
                    wt_hbm.at[:, pl.ds(0, A_CHUNK)], ins[p], isems[p]).wait()

                ihigh = jax.lax.bitwise_and(iota, 8)
                for j in range(8):
                    s7 = jax.lax.bitwise_and(iota + j, 7)
                    qa = jax.lax.bitwise_or(s7, ihigh)
                    qb = jax.lax.bitwise_xor(qa, 8)
                    cc = jax.lax.shift_left(s7, 4) + iota

                    def _blk(i, carry):
                        ga, gb = carry
                        va = plsc.load_gather(ins[p], [iota, ga])
                        vb = plsc.load_gather(ins[p], [iota, gb])
                        packed = plsc.bitcast(
                            plsc.pack(va, vb,
                                      format=plsc.PackFormat.INTERLEAVED),
                            jnp.float32)
                        plsc.store_scatter(
                            obs[p], [jnp.full((16,), i, jnp.int32), cc],
                            packed)
                        return (ga + 16, gb + 16)

                    plsc.parallel_loop(0, A_CHUNK // 16, 1, unroll=4,
                                       carry=(qa, qb))(_blk)
                pltpu.async_copy(obs[p], dst, osems[p])
                issue_in(t + 2, p)

        @pl.when(wid == 0)
        def _():
            pltpu.sync_copy(wtail_hbm, tailb)
            pltpu.sync_copy(tailb, scr_hbm.at[pl.ds(SCR_LINES - 4, 4), :])

        issue_in(0, 0)
        issue_in(1, 1)
        process(0, 0, True)
        process(1, 1, True)

        def outer(tt, carry):
            process(2 * tt, 0, False)
            process(2 * tt + 1, 1, False)
            return carry

        lax.fori_loop(1, A_TMAX // 2, outer, 0)

        dst0 = scr_hbm.at[pl.ds(0, A_LINES), :]
        pltpu.make_async_copy(ob0, dst0, osem0).wait()
        pltpu.make_async_copy(ob1, dst0, osem1).wait()

    return transpose_kernel


@functools.cache
def _build_gather():
    mesh = plsc.VectorSubcoreMesh(core_axis_name="c", subcore_axis_name="s")

    @functools.partial(
        pl.kernel,
        mesh=mesh,
        compiler_params=pltpu.CompilerParams(needs_layout_passes=False),
        out_type=jax.ShapeDtypeStruct((N_FIELDS, EMBED_DIM, BATCH),
                                      jnp.float32),
        scratch_types=[
            pltpu.VMEM((PER_WORKER,), jnp.int32),
            pltpu.VMEM((PER_WORKER,), jnp.int32),
            pltpu.VMEM((PER_WORKER,), jnp.int32),
            pltpu.VMEM((PER_WORKER,), jnp.int32),
            pltpu.VMEM((B_CHUNK, 128), jnp.float32),
            pltpu.VMEM((B_CHUNK, 128), jnp.float32),
            pltpu.VMEM((16, B_CHUNK), jnp.float32),
            pltpu.VMEM((16, B_CHUNK), jnp.float32),
            pltpu.SemaphoreType.DMA,
            pltpu.SemaphoreType.DMA,
            pltpu.SemaphoreType.DMA,
            pltpu.SemaphoreType.DMA,
        ],
    )
    def gather_kernel(scr_hbm, idx_hbm, out_hbm,
                      idxs_v, rv_all, sv_all, hv_all, st0, st1, ot0, ot1,
                      gsem0, gsem1, osem0, osem1):
        wid = lax.axis_index("s") * NUM_CORES + lax.axis_index("c")
        qbase = wid * PER_WORKER
        stages = (st0, st1)
        outs = (ot0, ot1)
        gsems = (gsem0, gsem1)
        osems = (osem0, osem1)
        iota = _iota16()

        pltpu.sync_copy(idx_hbm.at[pl.ds(qbase, PER_WORKER)], idxs_v)

        def _prep(k):
            v = idxs_v[pl.ds(k * 16, 16)]
            rv_all[pl.ds(k * 16, 16)] = jax.lax.shift_right_logical(v, 4)
            sv_all[pl.ds(k * 16, 16)] = jax.lax.shift_left(
                jax.lax.bitwise_and(v, 7), 4)
            hv_all[pl.ds(k * 16, 16)] = jax.lax.shift_left(
                jax.lax.bitwise_and(jax.lax.shift_right_logical(v, 3), 1),
                4)

        plsc.parallel_loop(0, PER_WORKER // 16, 1, unroll=4)(_prep)

        def fire(t, p):
            @pl.when(t < B_NCHUNKS)
            def _():
                toff = pl.multiple_of(t * B_CHUNK, B_CHUNK)
                pltpu.async_copy(
                    scr_hbm.at[rv_all.at[pl.ds(toff, B_CHUNK)]],
                    stages[p], gsems[p])

        def out_slab(t):
            q0 = qbase + t * B_CHUNK
            f = jax.lax.shift_right_logical(q0, 14)
            b0 = pl.multiple_of(jax.lax.bitwise_and(q0, BATCH - 1), B_CHUNK)
            return out_hbm.at[f, :, pl.ds(b0, B_CHUNK)]

        def process(t, p, first):
            dst = out_slab(t)
            if not first:
                pltpu.make_async_copy(outs[p], dst, osems[p]).wait()
            pltpu.make_async_copy(
                scr_hbm.at[rv_all.at[pl.ds(0, B_CHUNK)]],
                stages[p], gsems[p]).wait()

            toffv = jnp.full((16,), t * B_CHUNK, jnp.int32)
            hx = jax.lax.shift_left(jax.lax.bitwise_and(iota, 8), 1)
            for j in range(16):
                q = jax.lax.bitwise_and(iota + j, 15)

                def _blk(i, rv):
                    g = rv + toffv
                    sv = plsc.load_gather(sv_all, [g])
                    hv = plsc.load_gather(hv_all, [g])
                    w = plsc.load_gather(stages[p], [rv, sv + iota])
                    wi = plsc.bitcast(w, jnp.int32)
                    vals = plsc.bitcast(
                        jax.lax.shift_left(
                            jax.lax.shift_right_logical(
                                wi, jax.lax.bitwise_xor(hv, hx)), 16),
                        jnp.float32)
                    plsc.store_scatter(outs[p], [iota, rv], vals)
                    return rv + 16

                plsc.parallel_loop(0, B_CHUNK // 16, 1, unroll=4,
                                   carry=q)(_blk)
            pltpu.async_copy(outs[p], dst, osems[p])
            fire(t + 2, p)

        fire(0, 0)
        fire(1, 1)
        process(0, 0, True)
        process(1, 1, True)

        def outer(tt, carry):
            process(2 * tt, 0, False)
            process(2 * tt + 1, 1, False)
            return carry

        lax.fori_loop(1, B_NCHUNKS // 2, outer, 0)

        dst0 = out_hbm.at[0, :, pl.ds(0, B_CHUNK)]
        pltpu.make_async_copy(ot0, dst0, osem0).wait()
        pltpu.make_async_copy(ot1, dst0, osem1).wait()

    return gather_kernel


def kernel(x, W):
    wt = W.T
    wtail = lax.slice(W, (ROWS_MAIN, 0), (FEATURE_DIM, EMBED_DIM))
    wtail = wtail.astype(jnp.bfloat16).reshape(4, 2, 8, EMBED_DIM)
    wtail = jnp.transpose(wtail, (0, 2, 3, 1))
    flip = (jnp.arange(EMBED_DIM) >= 8)[None, None, :, None]
    wtail = jnp.where(flip, wtail[..., ::-1], wtail)
    wtail = jax.lax.bitcast_convert_type(wtail, jnp.float32)
    wtail = wtail.reshape(4, 128)
    idx = x.T.reshape(TOTAL).astype(jnp.int32)
    w_scr = _build_transpose()(wt, wtail)
    out3 = _build_gather()(w_scr, idx)
    return jnp.transpose(out3, (2, 0, 1))
